# baseline (speedup 1.0000x reference)
import jax
import jax.numpy as jnp
from jax.experimental import pallas as pl

K_SEL = 21
Q = 1024
KRUN = 32
TILE = 2048


def _stage_desc(v, i, dist):
    n, t = v.shape
    vr = v.reshape(n // (2 * dist), 2, dist, t)
    ir = i.reshape(n // (2 * dist), 2, dist, t)
    sel = vr[:, 0] >= vr[:, 1]
    fv = jnp.where(sel, vr[:, 0], vr[:, 1])
    fi = jnp.where(sel, ir[:, 0], ir[:, 1])
    sv = jnp.where(sel, vr[:, 1], vr[:, 0])
    si = jnp.where(sel, ir[:, 1], ir[:, 0])
    v = jnp.stack([fv, sv], axis=1).reshape(n, t)
    i = jnp.stack([fi, si], axis=1).reshape(n, t)
    return v, i


def _neg_bit(v, b):
    n, t = v.shape
    vr = v.reshape(n >> (b + 1), 2, 1 << b, t)
    return jnp.concatenate([vr[:, :1], -vr[:, 1:]], axis=1).reshape(n, t)


def _neg_xor(v, x, y):
    n, t = v.shape
    vr = v.reshape(n >> (x + 1), 2, 1 << (x - y - 1), 2, 1 << y, t)
    a = vr[:, 0]
    b = vr[:, 1]
    a = jnp.concatenate([a[:, :, :1], -a[:, :, 1:]], axis=2)
    b = jnp.concatenate([-b[:, :, :1], b[:, :, 1:]], axis=2)
    return jnp.stack([a, b], axis=1).reshape(n, t)


def _topk_body(xq_ref, xbt_ref, out_ref):
    s = jnp.dot(xq_ref[...], xbt_ref[...],
                preferred_element_type=jnp.float32)
    t = s.shape[-1]
    i = jax.lax.broadcasted_iota(jnp.int32, (Q, t), 0)

    v = _neg_xor(s, 6, 4)
    k = 2
    while k <= KRUN:
        d = k // 2
        while d >= 1:
            v, i = _stage_desc(v, i, d * KRUN)
            d //= 2
        if k < KRUN:
            kb = 5 + k.bit_length() - 1
            if 2 * k < KRUN:
                v = _neg_xor(v, kb + 1, kb)
            else:
                v = _neg_bit(v, kb)
        k *= 2

    sruns = KRUN
    while sruns > 1:
        n = v.shape[0]
        vr = v.reshape(n // sruns, 2, sruns // 2, t)
        ir = i.reshape(n // sruns, 2, sruns // 2, t)
        nb = -vr[:, 1]
        m = vr[:, 0] >= nb
        v = jnp.where(m, vr[:, 0], nb).reshape(n // 2, t)
        i = jnp.where(m, ir[:, 0], ir[:, 1]).reshape(n // 2, t)
        sruns //= 2
        if sruns > 1:
            v = _neg_bit(v, sruns.bit_length() - 2)
        d = KRUN // 2
        while d >= 1:
            v, i = _stage_desc(v, i, d * sruns)
            d //= 2

    for off in (0, 1):
        va = v[off:off + 30].reshape(15, 2, t)
        ia = i[off:off + 30].reshape(15, 2, t)
        beats = (va[:, 0] > va[:, 1]) | (
            (va[:, 0] == va[:, 1]) & (ia[:, 0] < ia[:, 1]))
        hv = jnp.where(beats, va[:, 0], va[:, 1])
        hi = jnp.where(beats, ia[:, 0], ia[:, 1])
        lv = jnp.where(beats, va[:, 1], va[:, 0])
        li = jnp.where(beats, ia[:, 1], ia[:, 0])
        vm = jnp.stack([hv, lv], axis=1).reshape(30, t)
        im = jnp.stack([hi, li], axis=1).reshape(30, t)
        vparts = [vm, v[off + 30:]] if off == 0 else [v[:off], vm, v[off + 30:]]
        iparts = [im, i[off + 30:]] if off == 0 else [i[:off], im, i[off + 30:]]
        v = jnp.concatenate(vparts, axis=0)
        i = jnp.concatenate(iparts, axis=0)

    out_ref[...] = i[:K_SEL]


def kernel(xq, xb):
    n = xb.shape[0]
    n_pad = ((n + TILE - 1) // TILE) * TILE
    xbt = jnp.pad(xb, ((0, n_pad - n), (0, 0))).T

    out = pl.pallas_call(
        _topk_body,
        grid=(n_pad // TILE,),
        in_specs=[
            pl.BlockSpec((Q, 16), lambda j: (0, 0)),
            pl.BlockSpec((16, TILE), lambda j: (0, j)),
        ],
        out_specs=pl.BlockSpec((K_SEL, TILE), lambda j: (0, j)),
        out_shape=jax.ShapeDtypeStruct((K_SEL, n_pad), jnp.int32),
    )(xq, xbt)
    return out[:, :n]


# speedup vs baseline: 1.1255x; 1.1255x over previous
import jax
import jax.numpy as jnp
from jax.experimental import pallas as pl

K_SEL = 21
Q = 1024
KRUN = 32
TILE = 1024


def _stage_desc(v, i, dist):
    n, t = v.shape
    vr = v.reshape(n // (2 * dist), 2, dist, t)
    ir = i.reshape(n // (2 * dist), 2, dist, t)
    sel = vr[:, 0] >= vr[:, 1]
    fv = jnp.where(sel, vr[:, 0], vr[:, 1])
    fi = jnp.where(sel, ir[:, 0], ir[:, 1])
    sv = jnp.where(sel, vr[:, 1], vr[:, 0])
    si = jnp.where(sel, ir[:, 1], ir[:, 0])
    v = jnp.stack([fv, sv], axis=1).reshape(n, t)
    i = jnp.stack([fi, si], axis=1).reshape(n, t)
    return v, i


def _stage2_desc(v, i, dist):
    n, t = v.shape
    vr = v.reshape(n // (4 * dist), 4, dist, t)
    ir = i.reshape(n // (4 * dist), 4, dist, t)
    va, vb, vc, vd = vr[:, 0], vr[:, 1], vr[:, 2], vr[:, 3]
    ia, ib, ic, id_ = ir[:, 0], ir[:, 1], ir[:, 2], ir[:, 3]
    s1 = va >= vc
    va, vc = jnp.where(s1, va, vc), jnp.where(s1, vc, va)
    ia, ic = jnp.where(s1, ia, ic), jnp.where(s1, ic, ia)
    s2 = vb >= vd
    vb, vd = jnp.where(s2, vb, vd), jnp.where(s2, vd, vb)
    ib, id_ = jnp.where(s2, ib, id_), jnp.where(s2, id_, ib)
    s3 = va >= vb
    va, vb = jnp.where(s3, va, vb), jnp.where(s3, vb, va)
    ia, ib = jnp.where(s3, ia, ib), jnp.where(s3, ib, ia)
    s4 = vc >= vd
    vc, vd = jnp.where(s4, vc, vd), jnp.where(s4, vd, vc)
    ic, id_ = jnp.where(s4, ic, id_), jnp.where(s4, id_, ic)
    v = jnp.stack([va, vb, vc, vd], axis=1).reshape(n, t)
    i = jnp.stack([ia, ib, ic, id_], axis=1).reshape(n, t)
    return v, i


def _merge_desc(v, i, d0, sruns):
    d = d0
    while d >= 2:
        v, i = _stage2_desc(v, i, (d // 2) * sruns)
        d //= 4
    if d == 1:
        v, i = _stage_desc(v, i, sruns)
    return v, i


def _neg_bit(v, b):
    n, t = v.shape
    vr = v.reshape(n >> (b + 1), 2, 1 << b, t)
    return jnp.concatenate([vr[:, :1], -vr[:, 1:]], axis=1).reshape(n, t)


def _neg_xor(v, x, y):
    n, t = v.shape
    vr = v.reshape(n >> (x + 1), 2, 1 << (x - y - 1), 2, 1 << y, t)
    a = vr[:, 0]
    b = vr[:, 1]
    a = jnp.concatenate([a[:, :, :1], -a[:, :, 1:]], axis=2)
    b = jnp.concatenate([-b[:, :, :1], b[:, :, 1:]], axis=2)
    return jnp.stack([a, b], axis=1).reshape(n, t)


def _topk_body(xq_ref, xbt_ref, out_ref):
    s = jnp.dot(xq_ref[...], xbt_ref[...],
                preferred_element_type=jnp.float32)
    t = s.shape[-1]
    i = jax.lax.broadcasted_iota(jnp.int32, (Q, t), 0)

    v = _neg_xor(s, 6, 4)
    k = 2
    while k <= KRUN:
        v, i = _merge_desc(v, i, k // 2, KRUN)
        if k < KRUN:
            kb = 5 + k.bit_length() - 1
            if 2 * k < KRUN:
                v = _neg_xor(v, kb + 1, kb)
            else:
                v = _neg_bit(v, kb)
        k *= 2

    sruns = KRUN
    while sruns > 1:
        n = v.shape[0]
        vr = v.reshape(n // sruns, 2, sruns // 2, t)
        ir = i.reshape(n // sruns, 2, sruns // 2, t)
        nb = -vr[:, 1]
        m = vr[:, 0] >= nb
        v = jnp.where(m, vr[:, 0], nb).reshape(n // 2, t)
        i = jnp.where(m, ir[:, 0], ir[:, 1]).reshape(n // 2, t)
        sruns //= 2
        if sruns > 1:
            v = _neg_bit(v, sruns.bit_length() - 2)
        v, i = _merge_desc(v, i, KRUN // 2, sruns)

    for off in (0, 1):
        va = v[off:off + 30].reshape(15, 2, t)
        ia = i[off:off + 30].reshape(15, 2, t)
        beats = (va[:, 0] > va[:, 1]) | (
            (va[:, 0] == va[:, 1]) & (ia[:, 0] < ia[:, 1]))
        hv = jnp.where(beats, va[:, 0], va[:, 1])
        hi = jnp.where(beats, ia[:, 0], ia[:, 1])
        lv = jnp.where(beats, va[:, 1], va[:, 0])
        li = jnp.where(beats, ia[:, 1], ia[:, 0])
        vm = jnp.stack([hv, lv], axis=1).reshape(30, t)
        im = jnp.stack([hi, li], axis=1).reshape(30, t)
        vparts = [vm, v[off + 30:]] if off == 0 else [v[:off], vm, v[off + 30:]]
        iparts = [im, i[off + 30:]] if off == 0 else [i[:off], im, i[off + 30:]]
        v = jnp.concatenate(vparts, axis=0)
        i = jnp.concatenate(iparts, axis=0)

    out_ref[...] = i[:K_SEL]


def kernel(xq, xb):
    n = xb.shape[0]
    n_pad = ((n + TILE - 1) // TILE) * TILE
    xbt = jnp.pad(xb, ((0, n_pad - n), (0, 0))).T

    out = pl.pallas_call(
        _topk_body,
        grid=(n_pad // TILE,),
        in_specs=[
            pl.BlockSpec((Q, 16), lambda j: (0, 0)),
            pl.BlockSpec((16, TILE), lambda j: (0, j)),
        ],
        out_specs=pl.BlockSpec((K_SEL, TILE), lambda j: (0, j)),
        out_shape=jax.ShapeDtypeStruct((K_SEL, n_pad), jnp.int32),
    )(xq, xbt)
    return out[:, :n]


# speedup vs baseline: 1.2082x; 1.0734x over previous
import jax
import jax.numpy as jnp
from jax.experimental import pallas as pl

K_SEL = 21
Q = 1024
KRUN = 32
TILE = 1024


def _stage_desc(v, i, dist):
    n, t = v.shape
    vr = v.reshape(n // (2 * dist), 2, dist, t)
    ir = i.reshape(n // (2 * dist), 2, dist, t)
    sel = vr[:, 0] >= vr[:, 1]
    fv = jnp.where(sel, vr[:, 0], vr[:, 1])
    fi = jnp.where(sel, ir[:, 0], ir[:, 1])
    sv = jnp.where(sel, vr[:, 1], vr[:, 0])
    si = jnp.where(sel, ir[:, 1], ir[:, 0])
    v = jnp.stack([fv, sv], axis=1).reshape(n, t)
    i = jnp.stack([fi, si], axis=1).reshape(n, t)
    return v, i


def _stage2_desc(v, i, dist):
    n, t = v.shape
    vr = v.reshape(n // (4 * dist), 4, dist, t)
    ir = i.reshape(n // (4 * dist), 4, dist, t)
    va, vb, vc, vd = vr[:, 0], vr[:, 1], vr[:, 2], vr[:, 3]
    ia, ib, ic, id_ = ir[:, 0], ir[:, 1], ir[:, 2], ir[:, 3]
    s1 = va >= vc
    va, vc = jnp.where(s1, va, vc), jnp.where(s1, vc, va)
    ia, ic = jnp.where(s1, ia, ic), jnp.where(s1, ic, ia)
    s2 = vb >= vd
    vb, vd = jnp.where(s2, vb, vd), jnp.where(s2, vd, vb)
    ib, id_ = jnp.where(s2, ib, id_), jnp.where(s2, id_, ib)
    s3 = va >= vb
    va, vb = jnp.where(s3, va, vb), jnp.where(s3, vb, va)
    ia, ib = jnp.where(s3, ia, ib), jnp.where(s3, ib, ia)
    s4 = vc >= vd
    vc, vd = jnp.where(s4, vc, vd), jnp.where(s4, vd, vc)
    ic, id_ = jnp.where(s4, ic, id_), jnp.where(s4, id_, ic)
    v = jnp.stack([va, vb, vc, vd], axis=1).reshape(n, t)
    i = jnp.stack([ia, ib, ic, id_], axis=1).reshape(n, t)
    return v, i


def _cx(sv, si, a, b):
    m = sv[a] >= sv[b]
    sv[a], sv[b] = jnp.where(m, sv[a], sv[b]), jnp.where(m, sv[b], sv[a])
    si[a], si[b] = jnp.where(m, si[a], si[b]), jnp.where(m, si[b], si[a])


def _stage3_desc(v, i, dist):
    n, t = v.shape
    vr = v.reshape(n // (8 * dist), 8, dist, t)
    ir = i.reshape(n // (8 * dist), 8, dist, t)
    sv = [vr[:, j] for j in range(8)]
    si = [ir[:, j] for j in range(8)]
    for a, b in ((0, 4), (1, 5), (2, 6), (3, 7),
                 (0, 2), (1, 3), (4, 6), (5, 7),
                 (0, 1), (2, 3), (4, 5), (6, 7)):
        _cx(sv, si, a, b)
    v = jnp.stack(sv, axis=1).reshape(n, t)
    i = jnp.stack(si, axis=1).reshape(n, t)
    return v, i


def _merge_desc(v, i, d0, sruns):
    d = d0
    while d >= 4:
        v, i = _stage3_desc(v, i, (d // 4) * sruns)
        d //= 8
    if d == 2:
        v, i = _stage2_desc(v, i, sruns)
    elif d == 1:
        v, i = _stage_desc(v, i, sruns)
    return v, i


def _neg_bit(v, b):
    n, t = v.shape
    vr = v.reshape(n >> (b + 1), 2, 1 << b, t)
    return jnp.concatenate([vr[:, :1], -vr[:, 1:]], axis=1).reshape(n, t)


def _neg_xor(v, x, y):
    n, t = v.shape
    vr = v.reshape(n >> (x + 1), 2, 1 << (x - y - 1), 2, 1 << y, t)
    a = vr[:, 0]
    b = vr[:, 1]
    a = jnp.concatenate([a[:, :, :1], -a[:, :, 1:]], axis=2)
    b = jnp.concatenate([-b[:, :, :1], b[:, :, 1:]], axis=2)
    return jnp.stack([a, b], axis=1).reshape(n, t)


def _topk_body(xq_ref, xbt_ref, out_ref):
    s = jnp.dot(xq_ref[...], xbt_ref[...],
                preferred_element_type=jnp.float32)
    t = s.shape[-1]
    i = jax.lax.broadcasted_iota(jnp.int32, (Q, t), 0)

    v = _neg_xor(s, 6, 4)
    k = 2
    while k <= KRUN:
        v, i = _merge_desc(v, i, k // 2, KRUN)
        if k < KRUN:
            kb = 5 + k.bit_length() - 1
            if 2 * k < KRUN:
                v = _neg_xor(v, kb + 1, kb)
            else:
                v = _neg_bit(v, kb)
        k *= 2

    sruns = KRUN
    while sruns > 1:
        n = v.shape[0]
        vr = v.reshape(n // sruns, 2, sruns // 2, t)
        ir = i.reshape(n // sruns, 2, sruns // 2, t)
        nb = -vr[:, 1]
        m = vr[:, 0] >= nb
        v = jnp.where(m, vr[:, 0], nb).reshape(n // 2, t)
        i = jnp.where(m, ir[:, 0], ir[:, 1]).reshape(n // 2, t)
        sruns //= 2
        if sruns > 1:
            v = _neg_bit(v, sruns.bit_length() - 2)
        v, i = _merge_desc(v, i, KRUN // 2, sruns)

    for off in (0, 1):
        va = v[off:off + 30].reshape(15, 2, t)
        ia = i[off:off + 30].reshape(15, 2, t)
        beats = (va[:, 0] > va[:, 1]) | (
            (va[:, 0] == va[:, 1]) & (ia[:, 0] < ia[:, 1]))
        hv = jnp.where(beats, va[:, 0], va[:, 1])
        hi = jnp.where(beats, ia[:, 0], ia[:, 1])
        lv = jnp.where(beats, va[:, 1], va[:, 0])
        li = jnp.where(beats, ia[:, 1], ia[:, 0])
        vm = jnp.stack([hv, lv], axis=1).reshape(30, t)
        im = jnp.stack([hi, li], axis=1).reshape(30, t)
        vparts = [vm, v[off + 30:]] if off == 0 else [v[:off], vm, v[off + 30:]]
        iparts = [im, i[off + 30:]] if off == 0 else [i[:off], im, i[off + 30:]]
        v = jnp.concatenate(vparts, axis=0)
        i = jnp.concatenate(iparts, axis=0)

    out_ref[...] = i[:K_SEL]


def kernel(xq, xb):
    n = xb.shape[0]
    n_pad = ((n + TILE - 1) // TILE) * TILE
    xbt = jnp.pad(xb, ((0, n_pad - n), (0, 0))).T

    out = pl.pallas_call(
        _topk_body,
        grid=(n_pad // TILE,),
        in_specs=[
            pl.BlockSpec((Q, 16), lambda j: (0, 0)),
            pl.BlockSpec((16, TILE), lambda j: (0, j)),
        ],
        out_specs=pl.BlockSpec((K_SEL, TILE), lambda j: (0, j)),
        out_shape=jax.ShapeDtypeStruct((K_SEL, n_pad), jnp.int32),
    )(xq, xbt)
    return out[:, :n]
